# baseline (speedup 1.0000x reference)
import functools

import jax
import jax.numpy as jnp
from jax.experimental import pallas as pl
from jax.experimental.pallas import tpu as pltpu

_NK = 2


def _w_chunk_copy(w_hbm, w_vmem, sems, kc, bk):
    return pltpu.make_async_copy(
        w_hbm.at[pl.ds(kc * bk, bk), :],
        w_vmem.at[pl.ds(kc * bk, bk), :],
        sems.at[kc],
    )


def _linear_kernel(x_ref, w_hbm, b_ref, o_ref, w_vmem, sems):
    t = pl.program_id(1)
    k = w_vmem.shape[0]
    bk = k // _NK

    @pl.when(t == 0)
    def _first_step():
        for kc in range(_NK):
            _w_chunk_copy(w_hbm, w_vmem, sems, kc, bk).start()
        o_ref[...] = jnp.broadcast_to(b_ref[...], o_ref.shape)
        for kc in range(_NK):
            _w_chunk_copy(w_hbm, w_vmem, sems, kc, bk).wait()
            o_ref[...] += jnp.dot(
                x_ref[:, kc * bk:(kc + 1) * bk],
                w_vmem[kc * bk:(kc + 1) * bk, :],
                preferred_element_type=jnp.float32,
            )

    @pl.when(t != 0)
    def _steady_step():
        o_ref[...] = (
            jnp.dot(x_ref[...], w_vmem[...], preferred_element_type=jnp.float32)
            + b_ref[...]
        )


@functools.partial(jax.jit, static_argnames=("num_cores", "block_m"))
def _linear(x2d, weight, bias, *, num_cores, block_m):
    m, k = x2d.shape
    _, n = weight.shape
    steps = m // (num_cores * block_m)

    return pl.pallas_call(
        _linear_kernel,
        out_shape=jax.ShapeDtypeStruct((m, n), jnp.float32),
        grid=(num_cores, steps),
        in_specs=[
            pl.BlockSpec((block_m, k), lambda i, t, s=steps: (i * s + t, 0)),
            pl.BlockSpec(memory_space=pl.ANY),
            pl.BlockSpec((1, n), lambda i, t: (0, 0)),
        ],
        out_specs=pl.BlockSpec((block_m, n), lambda i, t, s=steps: (i * s + t, 0)),
        scratch_shapes=[
            pltpu.VMEM((k, n), jnp.float32),
            pltpu.SemaphoreType.DMA((_NK,)),
        ],
        compiler_params=pltpu.CompilerParams(
            dimension_semantics=("parallel", "arbitrary"),
            vmem_limit_bytes=60 << 20,
        ),
        cost_estimate=pl.CostEstimate(
            flops=2 * m * k * n,
            transcendentals=0,
            bytes_accessed=4 * (m * k + k * n + m * n + n),
        ),
    )(x2d, weight, bias.reshape(1, n))


def kernel(x, weight, bias):
    orig_shape = x.shape
    in_features, out_features = weight.shape
    x2d = x.reshape(-1, in_features).astype(jnp.float32)
    out = _linear(
        x2d,
        weight.astype(jnp.float32),
        bias.astype(jnp.float32),
        num_cores=2,
        block_m=512,
    )
    return out.reshape(*orig_shape[:-1], out_features)


# speedup vs baseline: 1.0762x; 1.0762x over previous
import functools

import jax
import jax.numpy as jnp
from jax.experimental import pallas as pl
from jax.experimental.pallas import tpu as pltpu


def _linear_kernel(x_ref, w_ref, b_ref, o_hbm, obuf0, obuf1, sems, *, steps):
    i = pl.program_id(0)
    t = pl.program_id(1)
    bm = x_ref.shape[0]
    row_start = (i * steps + t) * bm
    p = jax.lax.rem(t, 2)

    for slot, buf, other in ((0, obuf0, obuf1), (1, obuf1, obuf0)):

        @pl.when(p == slot)
        def _(slot=slot, buf=buf, other=other):
            @pl.when(t >= 2)
            def _():
                pltpu.make_async_copy(buf, buf, sems.at[slot]).wait()

            buf[...] = (
                jnp.dot(x_ref[...], w_ref[...], preferred_element_type=jnp.float32)
                + b_ref[...]
            )
            pltpu.make_async_copy(
                buf, o_hbm.at[pl.ds(row_start, bm), :], sems.at[slot]
            ).start()

            @pl.when(t == steps - 1)
            def _():
                @pl.when(steps > 1)
                def _():
                    pltpu.make_async_copy(other, other, sems.at[1 - slot]).wait()

                pltpu.make_async_copy(buf, buf, sems.at[slot]).wait()


@functools.partial(jax.jit, static_argnames=("num_cores", "block_m"))
def _linear(x2d, weight, bias, *, num_cores, block_m):
    m, k = x2d.shape
    _, n = weight.shape
    steps = m // (num_cores * block_m)

    return pl.pallas_call(
        functools.partial(_linear_kernel, steps=steps),
        out_shape=jax.ShapeDtypeStruct((m, n), jnp.float32),
        grid=(num_cores, steps),
        in_specs=[
            pl.BlockSpec((block_m, k), lambda i, t, s=steps: (i * s + t, 0)),
            pl.BlockSpec((k, n), lambda i, t: (0, 0)),
            pl.BlockSpec((1, n), lambda i, t: (0, 0)),
        ],
        out_specs=pl.BlockSpec(memory_space=pl.ANY),
        scratch_shapes=[
            pltpu.VMEM((block_m, n), jnp.float32),
            pltpu.VMEM((block_m, n), jnp.float32),
            pltpu.SemaphoreType.DMA((2,)),
        ],
        compiler_params=pltpu.CompilerParams(
            dimension_semantics=("parallel", "arbitrary"),
            vmem_limit_bytes=60 << 20,
        ),
        cost_estimate=pl.CostEstimate(
            flops=2 * m * k * n,
            transcendentals=0,
            bytes_accessed=4 * (m * k + k * n + m * n + n),
        ),
    )(x2d, weight, bias.reshape(1, n))


def kernel(x, weight, bias):
    orig_shape = x.shape
    in_features, out_features = weight.shape
    x2d = x.reshape(-1, in_features).astype(jnp.float32)
    out = _linear(
        x2d,
        weight.astype(jnp.float32),
        bias.astype(jnp.float32),
        num_cores=2,
        block_m=512,
    )
    return out.reshape(*orig_shape[:-1], out_features)


# speedup vs baseline: 1.1221x; 1.0427x over previous
import functools

import jax
import jax.numpy as jnp
from jax.experimental import pallas as pl
from jax.experimental.pallas import tpu as pltpu


def _linear_block_kernel(x_ref, w_ref, b_ref, o_ref):
    o_ref[...] = (
        jnp.dot(x_ref[...], w_ref[...], preferred_element_type=jnp.float32)
        + b_ref[...]
    )


@functools.partial(jax.jit, static_argnames=("block_m",))
def _linear(x2d, weight, bias, *, block_m):
    m, k = x2d.shape
    _, n = weight.shape
    grid = (m // block_m,)

    return pl.pallas_call(
        _linear_block_kernel,
        out_shape=jax.ShapeDtypeStruct((m, n), jnp.float32),
        grid=grid,
        in_specs=[
            pl.BlockSpec((block_m, k), lambda i: (i, 0)),
            pl.BlockSpec((k, n), lambda i: (0, 0)),
            pl.BlockSpec((1, n), lambda i: (0, 0)),
        ],
        out_specs=pl.BlockSpec((block_m, n), lambda i: (i, 0)),
        compiler_params=pltpu.CompilerParams(
            dimension_semantics=("parallel",),
            vmem_limit_bytes=60 << 20,
        ),
        cost_estimate=pl.CostEstimate(
            flops=2 * m * k * n,
            transcendentals=0,
            bytes_accessed=4 * (m * k + k * n + m * n + n),
        ),
    )(x2d, weight, bias.reshape(1, n))


def kernel(x, weight, bias):
    orig_shape = x.shape
    in_features, out_features = weight.shape
    x2d = x.reshape(-1, in_features).astype(jnp.float32)
    out = _linear(
        x2d,
        weight.astype(jnp.float32),
        bias.astype(jnp.float32),
        block_m=512,
    )
    return out.reshape(*orig_shape[:-1], out_features)
